# initial kernel scaffold (unmeasured)
import jax
import jax.numpy as jnp
from jax import lax
from jax.experimental import pallas as pl
from jax.experimental.pallas import tpu as pltpu

B, H, D, BS = 8, 8, 128, 16
Y = 4
NEG = -1e30


def kernel(Q, K, V, bt, lens):
    P_loc = K.shape[0]
    T_loc = P_loc * BS
    NB = bt.shape[1]
    C = D + 128

    def body(q_ref, k_ref, v_ref, bt_ref, lens_ref, out_ref,
             part_ref, gather_ref, kh_ref, vh_ref,
             copy_sems, send_sems, recv_sems):
        my_x = lax.axis_index("x")
        my_y = lax.axis_index("y")
        my_z = lax.axis_index("z")
        left = (my_y - 1) % Y
        right = (my_y + 1) % Y

        barrier = pltpu.get_barrier_semaphore()
        pl.semaphore_signal(barrier, inc=1, device_id=(my_x, left, my_z),
                            device_id_type=pl.DeviceIdType.MESH)
        pl.semaphore_signal(barrier, inc=1, device_id=(my_x, right, my_z),
                            device_id_type=pl.DeviceIdType.MESH)
        pl.semaphore_wait(barrier, 2)

        bt3 = bt_ref[...].reshape(B, 1, NB)
        gid3 = (lax.broadcasted_iota(jnp.int32, (1, P_loc, 1), 1)
                + my_y * P_loc)
        slot3 = lax.broadcasted_iota(jnp.int32, (1, 1, NB), 2)
        lens3 = lens_ref[...].reshape(B, 1, 1)
        hit = jnp.logical_and(bt3 == gid3, slot3 < lens3)
        mult_pb = jnp.sum(jnp.where(hit, 1.0, 0.0), axis=2)

        tok_page = lax.broadcasted_iota(jnp.int32, (T_loc, 1), 0) // BS
        col = lax.broadcasted_iota(jnp.int32, (1, P_loc), 1)
        E = (tok_page == col).astype(jnp.float32)
        mult_tok = lax.dot_general(
            mult_pb, E, (((1,), (1,)), ((), ())),
            preferred_element_type=jnp.float32)
        alive = mult_tok > 0.5

        qs = q_ref[:, 0, :, :] * (D ** -0.5)

        for h in range(H):
            kc = pltpu.make_async_copy(
                k_ref.at[:, :, h, :], kh_ref, copy_sems.at[0])
            vc = pltpu.make_async_copy(
                v_ref.at[:, :, h, :], vh_ref, copy_sems.at[1])
            kc.start()
            vc.start()
            kc.wait()
            vc.wait()

            k_h = kh_ref[...].reshape(T_loc, D)
            v_h = vh_ref[...].reshape(T_loc, D)
            s = lax.dot_general(
                qs[:, h, :], k_h, (((1,), (1,)), ((), ())),
                preferred_element_type=jnp.float32)
            masked = jnp.where(alive, s, NEG)
            m = jnp.max(masked, axis=1, keepdims=True)
            p = jnp.exp(masked - m) * mult_tok
            l = jnp.sum(p, axis=1, keepdims=True)
            o = lax.dot_general(
                p, v_h, (((1,), (0,)), ((), ())),
                preferred_element_type=jnp.float32)
            part_ref[:, h, 0:D] = o
            part_ref[:, h, D:D + 1] = m
            part_ref[:, h, D + 1:D + 2] = l

        for h in range(Y - 1):
            src = part_ref if h == 0 else gather_ref.at[h - 1]
            rdma = pltpu.make_async_remote_copy(
                src_ref=src,
                dst_ref=gather_ref.at[h],
                send_sem=send_sems.at[h],
                recv_sem=recv_sems.at[h],
                device_id=(my_x, right, my_z),
                device_id_type=pl.DeviceIdType.MESH,
            )
            rdma.start()
            rdma.wait()

        chunks = [part_ref[...]] + [gather_ref[k] for k in range(Y - 1)]
        m_g = chunks[0][:, :, D:D + 1]
        for c in chunks[1:]:
            m_g = jnp.maximum(m_g, c[:, :, D:D + 1])
        acc_o = jnp.zeros((B, H, D), jnp.float32)
        acc_l = jnp.zeros((B, H, 1), jnp.float32)
        for c in chunks:
            w = jnp.exp(c[:, :, D:D + 1] - m_g)
            acc_o = acc_o + c[:, :, 0:D] * w
            acc_l = acc_l + c[:, :, D + 1:D + 2] * w
        out_ref[:, 0, :, :] = acc_o / acc_l

    return pl.pallas_call(
        body,
        out_shape=jax.ShapeDtypeStruct((B, 1, H, D), jnp.float32),
        in_specs=[
            pl.BlockSpec(memory_space=pltpu.VMEM),
            pl.BlockSpec(memory_space=pltpu.ANY),
            pl.BlockSpec(memory_space=pltpu.ANY),
            pl.BlockSpec(memory_space=pltpu.VMEM),
            pl.BlockSpec(memory_space=pltpu.VMEM),
        ],
        out_specs=pl.BlockSpec(memory_space=pltpu.VMEM),
        scratch_shapes=[
            pltpu.VMEM((B, H, D + 128), jnp.float32),
            pltpu.VMEM((Y - 1, B, H, D + 128), jnp.float32),
            pltpu.VMEM((K.shape[0], BS, D), jnp.float32),
            pltpu.VMEM((K.shape[0], BS, D), jnp.float32),
            pltpu.SemaphoreType.DMA((2,)),
            pltpu.SemaphoreType.DMA((Y - 1,)),
            pltpu.SemaphoreType.DMA((Y - 1,)),
        ],
        compiler_params=pltpu.CompilerParams(collective_id=0),
    )(Q, K, V, bt, lens)


# baseline (device time: 53792 ns/iter reference)
import jax
import jax.numpy as jnp
from jax import lax
from jax.experimental import pallas as pl
from jax.experimental.pallas import tpu as pltpu

B, H, D, BS = 8, 8, 128, 16
Y = 4
NEG = -1e30


def kernel(Q, K, V, bt, lens):
    P_loc = K.shape[0]
    T_loc = P_loc * BS
    NB = bt.shape[1]
    C = D + 128

    def body(q_ref, k_ref, v_ref, bt_ref, lens_ref, out_ref,
             part_ref, gather_ref, kh_ref, vh_ref,
             copy_sems, send_sems, recv_sems):
        my_x = lax.axis_index("x")
        my_y = lax.axis_index("y")
        my_z = lax.axis_index("z")
        left = (my_y - 1) % Y
        right = (my_y + 1) % Y

        barrier = pltpu.get_barrier_semaphore()
        pl.semaphore_signal(barrier, inc=1, device_id=(my_x, left, my_z),
                            device_id_type=pl.DeviceIdType.MESH)
        pl.semaphore_signal(barrier, inc=1, device_id=(my_x, right, my_z),
                            device_id_type=pl.DeviceIdType.MESH)
        pl.semaphore_wait(barrier, 2)

        bt3 = bt_ref[...].reshape(B, 1, NB)
        gid3 = (lax.broadcasted_iota(jnp.int32, (1, P_loc, 1), 1)
                + my_y * P_loc)
        slot3 = lax.broadcasted_iota(jnp.int32, (1, 1, NB), 2)
        lens3 = lens_ref[...].reshape(B, 1, 1)
        hit = jnp.logical_and(bt3 == gid3, slot3 < lens3)
        mult_pb = jnp.sum(jnp.where(hit, 1.0, 0.0), axis=2)

        tok_page = lax.broadcasted_iota(jnp.int32, (T_loc, 1), 0) // BS
        col = lax.broadcasted_iota(jnp.int32, (1, P_loc), 1)
        E = (tok_page == col).astype(jnp.float32)
        mult_tok = lax.dot_general(
            mult_pb, E, (((1,), (1,)), ((), ())),
            preferred_element_type=jnp.float32)
        alive = mult_tok > 0.5

        qs = q_ref[:, 0, :, :] * (D ** -0.5)

        for h in range(H):
            kc = pltpu.make_async_copy(
                k_ref.at[:, :, h, :], kh_ref, copy_sems.at[0])
            vc = pltpu.make_async_copy(
                v_ref.at[:, :, h, :], vh_ref, copy_sems.at[1])
            kc.start()
            vc.start()
            kc.wait()
            vc.wait()

            k_h = kh_ref[...].reshape(T_loc, D)
            v_h = vh_ref[...].reshape(T_loc, D)
            s = lax.dot_general(
                qs[:, h, :], k_h, (((1,), (1,)), ((), ())),
                preferred_element_type=jnp.float32)
            masked = jnp.where(alive, s, NEG)
            m = jnp.max(masked, axis=1, keepdims=True)
            p = jnp.exp(masked - m) * mult_tok
            l = jnp.sum(p, axis=1, keepdims=True)
            o = lax.dot_general(
                p, v_h, (((1,), (0,)), ((), ())),
                preferred_element_type=jnp.float32)
            part_ref[:, h, 0:D] = o
            part_ref[:, h, D:D + 1] = m
            part_ref[:, h, D + 1:D + 2] = l

        for h in range(Y - 1):
            src = part_ref if h == 0 else gather_ref.at[h - 1]
            rdma = pltpu.make_async_remote_copy(
                src_ref=src,
                dst_ref=gather_ref.at[h],
                send_sem=send_sems.at[h],
                recv_sem=recv_sems.at[h],
                device_id=(my_x, right, my_z),
                device_id_type=pl.DeviceIdType.MESH,
            )
            rdma.start()
            rdma.wait()

        chunks = [part_ref[...]] + [gather_ref[k] for k in range(Y - 1)]
        m_g = chunks[0][:, :, D:D + 1]
        for c in chunks[1:]:
            m_g = jnp.maximum(m_g, c[:, :, D:D + 1])
        acc_o = jnp.zeros((B, H, D), jnp.float32)
        acc_l = jnp.zeros((B, H, 1), jnp.float32)
        for c in chunks:
            w = jnp.exp(c[:, :, D:D + 1] - m_g)
            acc_o = acc_o + c[:, :, 0:D] * w
            acc_l = acc_l + c[:, :, D + 1:D + 2] * w
        out_ref[:, 0, :, :] = acc_o / acc_l

    return pl.pallas_call(
        body,
        out_shape=jax.ShapeDtypeStruct((B, 1, H, D), jnp.float32),
        in_specs=[
            pl.BlockSpec(memory_space=pltpu.VMEM),
            pl.BlockSpec(memory_space=pl.ANY),
            pl.BlockSpec(memory_space=pl.ANY),
            pl.BlockSpec(memory_space=pltpu.VMEM),
            pl.BlockSpec(memory_space=pltpu.VMEM),
        ],
        out_specs=pl.BlockSpec(memory_space=pltpu.VMEM),
        scratch_shapes=[
            pltpu.VMEM((B, H, D + 128), jnp.float32),
            pltpu.VMEM((Y - 1, B, H, D + 128), jnp.float32),
            pltpu.VMEM((K.shape[0], BS, D), jnp.float32),
            pltpu.VMEM((K.shape[0], BS, D), jnp.float32),
            pltpu.SemaphoreType.DMA((2,)),
            pltpu.SemaphoreType.DMA((Y - 1,)),
            pltpu.SemaphoreType.DMA((Y - 1,)),
        ],
        compiler_params=pltpu.CompilerParams(collective_id=0),
    )(Q, K, V, bt, lens)
